# baseline (speedup 1.0000x reference)
import jax
import jax.numpy as jnp
from jax import lax
from jax.experimental import pallas as pl
from jax.experimental.pallas import tpu as pltpu
from jax.experimental.pallas import tpu_sc as plsc

NC = 2
NS = 16
NW = NC * NS
EB = 128
LG = 8


def _sc_aggregate(n_pad, d, bpt):
    rows_per_tile = n_pad // NS
    cchunk = EB
    nchunk = rows_per_tile // cchunk
    dw = d // 2
    hb = bpt // 2

    def body(x_hbm, src_hbm, dst_hbm, w_hbm, out_hbm,
             src_all, dst_all, w_all, rows0, rows1, msgsA, msgsB,
             accum, semE, semG0, semG1, semSA, semSB):
        c = lax.axis_index("c")
        s = lax.axis_index("s")
        wid = c * NS + s
        b0 = wid * bpt

        def fetch_half(h):
            o = b0 + h * hb
            pltpu.make_async_copy(src_hbm.at[pl.ds(o, hb), :], src_all, semE).start()
            pltpu.make_async_copy(dst_hbm.at[pl.ds(o, hb), :, :], dst_all, semE).start()
            pltpu.make_async_copy(w_hbm.at[pl.ds(o, hb), :], w_all, semE).start()

        def wait_half():
            pltpu.make_async_copy(src_hbm.at[pl.ds(b0, hb), :], src_all, semE).wait()
            pltpu.make_async_copy(dst_hbm.at[pl.ds(b0, hb), :, :], dst_all, semE).wait()
            pltpu.make_async_copy(w_hbm.at[pl.ds(b0, hb), :], w_all, semE).wait()

        fetch_half(0)

        zero = jnp.zeros((16,), jnp.float32)

        def zrow(r, carry):
            for j in range(LG):
                msgsA[r, pl.ds(j * 16, 16)] = zero
                msgsB[r, pl.ds(j * 16, 16)] = zero
            return carry

        lax.fori_loop(0, EB // 2, zrow, 0)
        for k in range(nchunk * 2):
            r0 = s * rows_per_tile + k * (cchunk // 2)
            zsrc = msgsA if k % 2 == 0 else msgsB
            pltpu.sync_copy(zsrc.at[pl.ds(0, cchunk // 2), :],
                            accum.at[pl.ds(r0, cchunk // 2), :])
        plsc.subcore_barrier()
        wait_half()

        def start_gather(j, rows_r, sem):
            pltpu.make_async_copy(x_hbm.at[src_all.at[j]], rows_r, sem).start()

        def wait_gather(j, rows_r, sem):
            pltpu.make_async_copy(x_hbm.at[src_all.at[j]], rows_r, sem).wait()

        mask_hi = jnp.full((16,), -65536, jnp.int32)

        def scale_half(j, h, rows_r, msgs_r):
            def gbody(g, carry):
                wv = w_all[j, pl.ds(pl.multiple_of(h * 64 + g * 16, 16), 16)]
                for el in range(16):
                    e64 = g * 16 + el
                    e128 = h * 64 + g * 16 + el
                    wb = lax.gather(
                        wv, jnp.full((16, 1), el, jnp.int32),
                        lax.GatherDimensionNumbers(
                            offset_dims=(), collapsed_slice_dims=(0,),
                            start_index_map=(0,)),
                        slice_sizes=(1,),
                        mode=lax.GatherScatterMode.PROMISE_IN_BOUNDS)
                    for k in range(dw // 16):
                        v = rows_r[e128, pl.ds(k * 16, 16)]
                        lo = lax.bitcast_convert_type(v << 16, jnp.float32)
                        hi = lax.bitcast_convert_type(v & mask_hi, jnp.float32)
                        msgs_r[e64, pl.ds(k * 32, 16)] = lo * wb
                        msgs_r[e64, pl.ds(k * 32 + 16, 16)] = hi * wb
                return carry

            lax.fori_loop(0, 4, gbody, 0)

        def start_scatter(j, h, msgs_r, sem):
            pltpu.async_copy(msgs_r, accum.at[dst_all.at[j, h]], sem, add=True)

        def wait_scatter(msgs_r, sem):
            pltpu.make_async_copy(msgs_r, accum.at[dst_all.at[0, 0]], sem).wait()

        def process(j, rows_r, first):
            if not first:
                wait_scatter(msgsA, semSA)
            scale_half(j, 0, rows_r, msgsA)
            start_scatter(j, 0, msgsA, semSA)
            if not first:
                wait_scatter(msgsB, semSB)
            scale_half(j, 1, rows_r, msgsB)
            start_scatter(j, 1, msgsB, semSB)

        pairs = hb // 2

        def loop_body(i, carry):
            j = 2 * i
            start_gather(j + 1, rows1, semG1)
            wait_gather(j, rows0, semG0)
            process(j, rows0, False)

            @pl.when(i < pairs - 1)
            def _():
                start_gather(j + 2, rows0, semG0)

            wait_gather(j + 1, rows1, semG1)
            process(j + 1, rows1, False)
            return carry

        for h in range(2):
            if h == 1:
                wait_scatter(msgsA, semSA)
                wait_scatter(msgsB, semSB)
                fetch_half(1)
                wait_half()
            start_gather(0, rows0, semG0)
            start_gather(1, rows1, semG1)
            wait_gather(0, rows0, semG0)
            process(0, rows0, True)

            @pl.when(pairs > 1)
            def _():
                start_gather(2, rows0, semG0)

            wait_gather(1, rows1, semG1)
            process(1, rows1, False)
            lax.fori_loop(1, pairs, loop_body, 0)
        wait_scatter(msgsA, semSA)
        wait_scatter(msgsB, semSB)
        plsc.subcore_barrier()

        for k in range(nchunk * 2):
            r0 = s * rows_per_tile + k * (cchunk // 2)
            wdst = msgsA if k % 2 == 0 else msgsB
            pltpu.sync_copy(accum.at[pl.ds(r0, cchunk // 2), :], wdst)
            pltpu.sync_copy(wdst, out_hbm.at[c, pl.ds(r0, cchunk // 2), :])

    mesh = plsc.VectorSubcoreMesh(core_axis_name="c", subcore_axis_name="s",
                                  num_cores=NC, num_subcores=NS)
    return pl.kernel(
        body,
        out_type=jax.ShapeDtypeStruct((NC, n_pad, d), jnp.float32),
        mesh=mesh,
        compiler_params=pltpu.CompilerParams(use_tc_tiling_on_sc=False),
        scratch_types=[
            pltpu.VMEM((hb, EB), jnp.int32),
            pltpu.VMEM((hb, 2, EB // 2), jnp.int32),
            pltpu.VMEM((hb, EB), jnp.float32),
            pltpu.VMEM((EB, dw), jnp.int32),
            pltpu.VMEM((EB, dw), jnp.int32),
            pltpu.VMEM((EB // 2, d), jnp.float32),
            pltpu.VMEM((EB // 2, d), jnp.float32),
            pltpu.VMEM_SHARED((n_pad, d), jnp.float32),
            pltpu.SemaphoreType.DMA,
            pltpu.SemaphoreType.DMA,
            pltpu.SemaphoreType.DMA,
            pltpu.SemaphoreType.DMA,
            pltpu.SemaphoreType.DMA,
        ],
    )


def _tc_body(p_ref, w_ref, b_ref, o_ref):
    acc = p_ref[0] + p_ref[1]
    o_ref[...] = (
        jnp.dot(acc, w_ref[...], preferred_element_type=jnp.float32)
        + b_ref[...]
    )


@jax.jit
def kernel(x, edge_index, edge_weight, W, b):
    n, d_in = x.shape
    d_out = W.shape[1]
    e = edge_weight.shape[0]

    src = edge_index[0].astype(jnp.int32)
    dst = edge_index[1].astype(jnp.int32)
    w = edge_weight.astype(jnp.float32)

    ept = EB * NW
    bpt = 4 * -(-e // (ept * 4))
    e_pad = bpt * ept
    n_pad = -(-n // (NS * EB)) * NS * EB
    npe = e_pad - e
    fill = jnp.arange(npe, dtype=jnp.int32)
    src = jnp.concatenate([src, fill % n]).reshape(-1, EB)
    dst = jnp.concatenate([dst, n + fill % (n_pad - n)]).reshape(-1, 2, EB // 2)
    w = jnp.concatenate([w, jnp.zeros((npe,), jnp.float32)]).reshape(-1, EB)

    xw = lax.bitcast_convert_type(
        x.astype(jnp.bfloat16).reshape(n, d_in // 2, 2), jnp.int32)

    partials = _sc_aggregate(n_pad, d_in, bpt)(xw, src, dst, w)

    midx = jnp.arange(d_in)
    grp, t = midx // 32, midx % 32
    perm = jnp.where(t < 16, 32 * grp + 2 * t, 32 * grp + 2 * (t - 16) + 1)
    W_perm = W[perm, :]

    rows_blk = 1000 if n % 1000 == 0 else n
    grid = n // rows_blk
    out = pl.pallas_call(
        _tc_body,
        grid=(grid,),
        in_specs=[
            pl.BlockSpec((NC, rows_blk, d_in), lambda i: (0, i, 0)),
            pl.BlockSpec((d_in, d_out), lambda i: (0, 0)),
            pl.BlockSpec((1, d_out), lambda i: (0, 0)),
        ],
        out_specs=pl.BlockSpec((rows_blk, d_out), lambda i: (i, 0)),
        out_shape=jax.ShapeDtypeStruct((n, d_out), jnp.float32),
    )(partials, W_perm, b.reshape(1, d_out))
    return out


# speedup vs baseline: 1.9987x; 1.9987x over previous
import functools

import jax
import jax.numpy as jnp
from jax import lax
from jax.experimental import pallas as pl
from jax.experimental.pallas import tpu as pltpu
from jax.experimental.pallas import tpu_sc as plsc

NC = 2
NS = 16
NW = NC * NS
EB = 128
LG = 8


def _sc_aggregate(n_pad, d, bpt):
    rows_per_tile = n_pad // NS
    cchunk = EB
    nchunk = rows_per_tile // cchunk

    def body(x_hbm, src_hbm, dst_hbm, w_hbm, out_hbm,
             src_all, dst_all, w_all, rows0, rows1,
             accum, semE, semG0, semG1):
        c = lax.axis_index("c")
        s = lax.axis_index("s")
        wid = c * NS + s

        hb = bpt // 2
        b0 = wid * bpt

        def fetch_half(h):
            o = b0 + h * hb
            pltpu.make_async_copy(src_hbm.at[pl.ds(o, hb), :], src_all, semE).start()
            pltpu.make_async_copy(dst_hbm.at[pl.ds(o, hb), :], dst_all, semE).start()
            pltpu.make_async_copy(w_hbm.at[pl.ds(o, hb), :], w_all, semE).start()

        def wait_half():
            pltpu.make_async_copy(src_hbm.at[pl.ds(b0, hb), :], src_all, semE).wait()
            pltpu.make_async_copy(dst_hbm.at[pl.ds(b0, hb), :], dst_all, semE).wait()
            pltpu.make_async_copy(w_hbm.at[pl.ds(b0, hb), :], w_all, semE).wait()

        fetch_half(0)

        zero = jnp.zeros((16,), jnp.float32)

        def zrow(r, carry):
            for j in range(LG):
                rows0[r, pl.ds(j * 16, 16)] = zero
            return carry

        lax.fori_loop(0, EB, zrow, 0)
        for k in range(nchunk):
            r0 = s * rows_per_tile + k * cchunk
            pltpu.sync_copy(rows0.at[pl.ds(0, cchunk), :],
                            accum.at[pl.ds(r0, cchunk), :])
        plsc.subcore_barrier()

        def start_gather(j, rows_r, sem):
            pltpu.make_async_copy(x_hbm.at[src_all.at[j]], rows_r, sem).start()

        def wait_gather(j, rows_r, sem):
            pltpu.make_async_copy(x_hbm.at[src_all.at[j]], rows_r, sem).wait()

        def scale(j, rows_r):
            def gbody(g, carry):
                wv = w_all[j, pl.ds(pl.multiple_of(g * 16, 16), 16)]
                for el in range(16):
                    e = g * 16 + el
                    wb = lax.gather(
                        wv, jnp.full((16, 1), el, jnp.int32),
                        lax.GatherDimensionNumbers(
                            offset_dims=(), collapsed_slice_dims=(0,),
                            start_index_map=(0,)),
                        slice_sizes=(1,),
                        mode=lax.GatherScatterMode.PROMISE_IN_BOUNDS)
                    for jj in range(LG):
                        sl = pl.ds(jj * 16, 16)
                        rows_r[e, sl] = rows_r[e, sl] * wb
                return carry

            lax.fori_loop(0, EB // 16, gbody, 0)

        def scatter_add(j, rows_r):
            pltpu.sync_copy(rows_r, accum.at[dst_all.at[j]], add=True)

        pairs = hb // 2

        def loop_body(i, carry):
            j = 2 * i
            start_gather(j + 1, rows1, semG1)
            wait_gather(j, rows0, semG0)
            scale(j, rows0)
            scatter_add(j, rows0)

            @pl.when(i < pairs - 1)
            def _():
                start_gather(j + 2, rows0, semG0)

            wait_gather(j + 1, rows1, semG1)
            scale(j + 1, rows1)
            scatter_add(j + 1, rows1)
            return carry

        for h in range(2):
            if h == 1:
                fetch_half(1)
            wait_half()
            start_gather(0, rows0, semG0)
            lax.fori_loop(0, pairs, loop_body, 0)
        plsc.subcore_barrier()

        for k in range(nchunk):
            r0 = s * rows_per_tile + k * cchunk
            pltpu.sync_copy(accum.at[pl.ds(r0, cchunk), :],
                            out_hbm.at[c, pl.ds(r0, cchunk), :])

    mesh = plsc.VectorSubcoreMesh(core_axis_name="c", subcore_axis_name="s",
                                  num_cores=NC, num_subcores=NS)
    return pl.kernel(
        body,
        out_type=jax.ShapeDtypeStruct((NC, n_pad, d), jnp.float32),
        mesh=mesh,
        scratch_types=[
            pltpu.VMEM((bpt // 2, EB), jnp.int32),
            pltpu.VMEM((bpt // 2, EB), jnp.int32),
            pltpu.VMEM((bpt // 2, EB), jnp.float32),
            pltpu.VMEM((EB, d), jnp.float32),
            pltpu.VMEM((EB, d), jnp.float32),
            pltpu.VMEM_SHARED((n_pad, d), jnp.float32),
            pltpu.SemaphoreType.DMA,
            pltpu.SemaphoreType.DMA,
            pltpu.SemaphoreType.DMA,
        ],
    )


def _tc_body(p_ref, w_ref, b_ref, o_ref):
    acc = p_ref[0] + p_ref[1]
    o_ref[...] = (
        jnp.dot(acc, w_ref[...], preferred_element_type=jnp.float32)
        + b_ref[...]
    )


@jax.jit
def kernel(x, edge_index, edge_weight, W, b):
    n, d_in = x.shape
    d_out = W.shape[1]
    e = edge_weight.shape[0]

    src = edge_index[0].astype(jnp.int32)
    dst = edge_index[1].astype(jnp.int32)
    w = edge_weight.astype(jnp.float32)

    ept = EB * NW
    bpt = 2 * -(-e // (ept * 2))
    e_pad = bpt * ept
    n_pad = -(-n // (NS * EB)) * NS * EB
    npe = e_pad - e
    fill = jnp.arange(npe, dtype=jnp.int32)
    src = jnp.concatenate([src, fill % n]).reshape(-1, EB)
    dst = jnp.concatenate([dst, n + fill % (n_pad - n)]).reshape(-1, EB)
    w = jnp.concatenate([w, jnp.zeros((npe,), jnp.float32)]).reshape(-1, EB)

    partials = _sc_aggregate(n_pad, d_in, bpt)(x, src, dst, w)

    rows_blk = 1000 if n % 1000 == 0 else n
    grid = n // rows_blk
    out = pl.pallas_call(
        _tc_body,
        grid=(grid,),
        in_specs=[
            pl.BlockSpec((NC, rows_blk, d_in), lambda i: (0, i, 0)),
            pl.BlockSpec((d_in, d_out), lambda i: (0, 0)),
            pl.BlockSpec((1, d_out), lambda i: (0, 0)),
        ],
        out_specs=pl.BlockSpec((rows_blk, d_out), lambda i: (i, 0)),
        out_shape=jax.ShapeDtypeStruct((n, d_out), jnp.float32),
    )(partials, W, b.reshape(1, d_out))
    return out


# speedup vs baseline: 2.0037x; 1.0025x over previous
import functools

import jax
import jax.numpy as jnp
from jax import lax
from jax.experimental import pallas as pl
from jax.experimental.pallas import tpu as pltpu
from jax.experimental.pallas import tpu_sc as plsc

NC = 2
NS = 16
NW = NC * NS
EB = 128
LG = 8


def _sc_aggregate(n, n_pad, nb_real, d, bpt):
    rows_per_tile = n_pad // NS
    cchunk = EB
    nchunk = rows_per_tile // cchunk
    hb = bpt // 2
    last_b0 = (NW - 1) * bpt
    real_h = [max(0, min(hb, nb_real - last_b0 - h * hb)) for h in range(2)]
    bulk_h = [r // 8 * 8 for r in real_h]
    tail_h = [r - b for r, b in zip(real_h, bulk_h)]
    TAIL = 8

    def body(x_hbm, src_hbm, dst_hbm, w_hbm, tsrc_hbm, tdst_hbm, tw_hbm,
             out_hbm, src_all, dst_all, w_all, rows0, rows1,
             accum, semE, semG0, semG1):
        c = lax.axis_index("c")
        s = lax.axis_index("s")
        wid = c * NS + s
        b0 = wid * bpt
        is_last = wid == NW - 1

        lanes = jax.lax.iota(jnp.int32, 16)
        wzero = jnp.zeros((16,), jnp.float32)

        def synth_fill(row_lo, nrows):
            def fr(r, carry):
                for g in range(LG):
                    sl = pl.ds(g * 16, 16)
                    t = lanes * 13 + (r * 128 + g * 16)
                    src_all[row_lo + r, sl] = t % n
                    dst_all[row_lo + r, sl] = n + (t % (n_pad - n))
                    w_all[row_lo + r, sl] = wzero
                return carry

            lax.fori_loop(0, nrows, fr, 0)

        def last_tail_copies(h):
            rb = bulk_h[h]
            o = b0 + h * hb
            cps = []
            if rb > 0:
                cps += [
                    pltpu.make_async_copy(
                        src_hbm.at[pl.ds(o, rb), :], src_all.at[pl.ds(0, rb), :], semE),
                    pltpu.make_async_copy(
                        dst_hbm.at[pl.ds(o, rb), :], dst_all.at[pl.ds(0, rb), :], semE),
                    pltpu.make_async_copy(
                        w_hbm.at[pl.ds(o, rb), :], w_all.at[pl.ds(0, rb), :], semE),
                ]
            if tail_h[h] > 0:
                cps += [
                    pltpu.make_async_copy(
                        tsrc_hbm, src_all.at[pl.ds(rb, TAIL), :], semE),
                    pltpu.make_async_copy(
                        tdst_hbm, dst_all.at[pl.ds(rb, TAIL), :], semE),
                    pltpu.make_async_copy(
                        tw_hbm, w_all.at[pl.ds(rb, TAIL), :], semE),
                ]
            return cps

        def fetch_half(h):
            o = b0 + h * hb

            @pl.when(jnp.logical_not(is_last))
            def _():
                pltpu.make_async_copy(src_hbm.at[pl.ds(o, hb), :], src_all, semE).start()
                pltpu.make_async_copy(dst_hbm.at[pl.ds(o, hb), :], dst_all, semE).start()
                pltpu.make_async_copy(w_hbm.at[pl.ds(o, hb), :], w_all, semE).start()

            @pl.when(is_last)
            def _():
                for cp in last_tail_copies(h):
                    cp.start()
                staged = bulk_h[h] + (TAIL if tail_h[h] > 0 else 0)
                if staged < hb:
                    synth_fill(staged, hb - staged)

        def wait_half(h):
            @pl.when(jnp.logical_not(is_last))
            def _():
                pltpu.make_async_copy(src_hbm.at[pl.ds(b0, hb), :], src_all, semE).wait()
                pltpu.make_async_copy(dst_hbm.at[pl.ds(b0, hb), :], dst_all, semE).wait()
                pltpu.make_async_copy(w_hbm.at[pl.ds(b0, hb), :], w_all, semE).wait()

            if real_h[h] > 0:
                @pl.when(is_last)
                def _():
                    for cp in last_tail_copies(h):
                        cp.wait()

        fetch_half(0)

        zero = jnp.zeros((16,), jnp.float32)

        def zrow(r, carry):
            for j in range(LG):
                rows0[r, pl.ds(j * 16, 16)] = zero
            return carry

        lax.fori_loop(0, EB, zrow, 0)
        for k in range(nchunk):
            r0 = s * rows_per_tile + k * cchunk
            pltpu.sync_copy(rows0.at[pl.ds(0, cchunk), :],
                            accum.at[pl.ds(r0, cchunk), :])
        plsc.subcore_barrier()

        def start_gather(j, rows_r, sem):
            pltpu.make_async_copy(x_hbm.at[src_all.at[j]], rows_r, sem).start()

        def wait_gather(j, rows_r, sem):
            pltpu.make_async_copy(x_hbm.at[src_all.at[j]], rows_r, sem).wait()

        def scale(j, rows_r):
            def gbody(g, carry):
                wv = w_all[j, pl.ds(pl.multiple_of(g * 16, 16), 16)]
                for el in range(16):
                    e = g * 16 + el
                    wb = lax.gather(
                        wv, jnp.full((16, 1), el, jnp.int32),
                        lax.GatherDimensionNumbers(
                            offset_dims=(), collapsed_slice_dims=(0,),
                            start_index_map=(0,)),
                        slice_sizes=(1,),
                        mode=lax.GatherScatterMode.PROMISE_IN_BOUNDS)
                    for jj in range(LG):
                        sl = pl.ds(jj * 16, 16)
                        rows_r[e, sl] = rows_r[e, sl] * wb
                return carry

            lax.fori_loop(0, EB // 16, gbody, 0)

        def scatter_add(j, rows_r):
            pltpu.sync_copy(rows_r, accum.at[dst_all.at[j]], add=True)

        pairs = hb // 2

        def loop_body(i, carry):
            j = 2 * i
            start_gather(j + 1, rows1, semG1)
            wait_gather(j, rows0, semG0)
            scale(j, rows0)
            scatter_add(j, rows0)

            @pl.when(i < pairs - 1)
            def _():
                start_gather(j + 2, rows0, semG0)

            wait_gather(j + 1, rows1, semG1)
            scale(j + 1, rows1)
            scatter_add(j + 1, rows1)
            return carry

        for h in range(2):
            if h == 1:
                fetch_half(1)
            wait_half(h)
            start_gather(0, rows0, semG0)
            lax.fori_loop(0, pairs, loop_body, 0)
        plsc.subcore_barrier()

        for k in range(nchunk):
            r0 = s * rows_per_tile + k * cchunk
            pltpu.sync_copy(accum.at[pl.ds(r0, cchunk), :],
                            out_hbm.at[c, pl.ds(r0, cchunk), :])

    mesh = plsc.VectorSubcoreMesh(core_axis_name="c", subcore_axis_name="s",
                                  num_cores=NC, num_subcores=NS)
    return pl.kernel(
        body,
        out_type=jax.ShapeDtypeStruct((NC, n_pad, d), jnp.float32),
        mesh=mesh,
        scratch_types=[
            pltpu.VMEM((bpt // 2, EB), jnp.int32),
            pltpu.VMEM((bpt // 2, EB), jnp.int32),
            pltpu.VMEM((bpt // 2, EB), jnp.float32),
            pltpu.VMEM((EB, d), jnp.float32),
            pltpu.VMEM((EB, d), jnp.float32),
            pltpu.VMEM_SHARED((n_pad, d), jnp.float32),
            pltpu.SemaphoreType.DMA,
            pltpu.SemaphoreType.DMA,
            pltpu.SemaphoreType.DMA,
        ],
    )


def _tc_body(p_ref, w_ref, b_ref, o_ref):
    acc = p_ref[0] + p_ref[1]
    o_ref[...] = (
        jnp.dot(acc, w_ref[...], preferred_element_type=jnp.float32)
        + b_ref[...]
    )


@jax.jit
def kernel(x, edge_index, edge_weight, W, b):
    n, d_in = x.shape
    d_out = W.shape[1]
    e = edge_weight.shape[0]

    src = edge_index[0].astype(jnp.int32)
    dst = edge_index[1].astype(jnp.int32)
    w = edge_weight.astype(jnp.float32)

    ept = EB * NW
    bpt = 2 * -(-e // (ept * 2))
    n_pad = -(-n // (NS * EB)) * NS * EB
    if e % EB != 0 or e // EB <= (NW - 1) * bpt:
        e_pad = bpt * ept
        npe = e_pad - e
        fill = jnp.arange(npe, dtype=jnp.int32)
        src = jnp.concatenate([src, fill % n])
        dst = jnp.concatenate([dst, n + fill % (n_pad - n)])
        w = jnp.concatenate([w, jnp.zeros((npe,), jnp.float32)])
        e = e_pad
    nb_real = e // EB

    t_blocks = nb_real - nb_real // 8 * 8
    if t_blocks > 0:
        toff = (nb_real - t_blocks) * EB
        t_pad = (8 - t_blocks) * EB
        tfill = jnp.arange(t_pad, dtype=jnp.int32)
        tsrc = jnp.concatenate([src[toff:], tfill % n]).reshape(8, EB)
        tdst = jnp.concatenate([dst[toff:], n + tfill % (n_pad - n)]).reshape(8, EB)
        tw = jnp.concatenate([w[toff:], jnp.zeros((t_pad,), jnp.float32)]).reshape(8, EB)
    else:
        tsrc = jnp.zeros((8, EB), jnp.int32)
        tdst = jnp.full((8, EB), n, jnp.int32)
        tw = jnp.zeros((8, EB), jnp.float32)

    partials = _sc_aggregate(n, n_pad, nb_real, d_in, bpt)(
        x, src.reshape(nb_real, EB), dst.reshape(nb_real, EB),
        w.reshape(nb_real, EB), tsrc, tdst, tw)

    rows_blk = 1000 if n % 1000 == 0 else n
    grid = n // rows_blk
    out = pl.pallas_call(
        _tc_body,
        grid=(grid,),
        in_specs=[
            pl.BlockSpec((NC, rows_blk, d_in), lambda i: (0, i, 0)),
            pl.BlockSpec((d_in, d_out), lambda i: (0, 0)),
            pl.BlockSpec((1, d_out), lambda i: (0, 0)),
        ],
        out_specs=pl.BlockSpec((rows_blk, d_out), lambda i: (i, 0)),
        out_shape=jax.ShapeDtypeStruct((n, d_out), jnp.float32),
    )(partials, W, b.reshape(1, d_out))
    return out


# speedup vs baseline: 2.0053x; 1.0008x over previous
import functools

import jax
import jax.numpy as jnp
from jax import lax
from jax.experimental import pallas as pl
from jax.experimental.pallas import tpu as pltpu
from jax.experimental.pallas import tpu_sc as plsc

NC = 2
NS = 16
NW = NC * NS
EB = 128
LG = 8


def _sc_aggregate(n, n_pad, nb_real, d, bpt):
    rows_per_tile = n_pad // NS
    cchunk = EB
    nchunk = rows_per_tile // cchunk
    hb = bpt // 2
    last_b0 = (NW - 1) * bpt
    real_h = [max(0, min(hb, nb_real - last_b0 - h * hb)) for h in range(2)]
    bulk_h = [r // 8 * 8 for r in real_h]
    tail_h = [r - b for r, b in zip(real_h, bulk_h)]
    TAIL = 8

    def body(x_hbm, src_hbm, dst_hbm, w_hbm, tsrc_hbm, tdst_hbm, tw_hbm,
             out_hbm, src_all, dst_all, w_all, rows0, rows1,
             accum, semE, semG0, semG1):
        c = lax.axis_index("c")
        s = lax.axis_index("s")
        wid = c * NS + s
        b0 = wid * bpt
        is_last = wid == NW - 1

        lanes = jax.lax.iota(jnp.int32, 16)
        wzero = jnp.zeros((16,), jnp.float32)

        def synth_fill(row_lo, nrows):
            def fr(r, carry):
                for g in range(LG):
                    sl = pl.ds(g * 16, 16)
                    fl = pl.ds(pl.multiple_of((row_lo + r) * EB + g * 16, 16), 16)
                    t = lanes * 13 + (r * 128 + g * 16)
                    src_all[fl] = t % n
                    dst_all[row_lo + r, sl] = n + (t % (n_pad - n))
                    w_all[fl] = wzero
                return carry

            lax.fori_loop(0, nrows, fr, 0)

        def last_tail_copies(h):
            rb = bulk_h[h]
            o = b0 + h * hb
            cps = []
            if rb > 0:
                cps += [
                    pltpu.make_async_copy(
                        src_hbm.at[pl.ds(o * EB, rb * EB)],
                        src_all.at[pl.ds(0, rb * EB)], semE),
                    pltpu.make_async_copy(
                        dst_hbm.at[pl.ds(o, rb), :], dst_all.at[pl.ds(0, rb), :], semE),
                    pltpu.make_async_copy(
                        w_hbm.at[pl.ds(o * EB, rb * EB)],
                        w_all.at[pl.ds(0, rb * EB)], semE),
                ]
            if tail_h[h] > 0:
                cps += [
                    pltpu.make_async_copy(
                        tsrc_hbm, src_all.at[pl.ds(rb * EB, TAIL * EB)], semE),
                    pltpu.make_async_copy(
                        tdst_hbm, dst_all.at[pl.ds(rb, TAIL), :], semE),
                    pltpu.make_async_copy(
                        tw_hbm, w_all.at[pl.ds(rb * EB, TAIL * EB)], semE),
                ]
            return cps

        def fetch_half(h):
            o = b0 + h * hb

            @pl.when(jnp.logical_not(is_last))
            def _():
                pltpu.make_async_copy(src_hbm.at[pl.ds(o * EB, hb * EB)], src_all, semE).start()
                pltpu.make_async_copy(dst_hbm.at[pl.ds(o, hb), :], dst_all, semE).start()
                pltpu.make_async_copy(w_hbm.at[pl.ds(o * EB, hb * EB)], w_all, semE).start()

            @pl.when(is_last)
            def _():
                for cp in last_tail_copies(h):
                    cp.start()
                staged = bulk_h[h] + (TAIL if tail_h[h] > 0 else 0)
                if staged < hb:
                    synth_fill(staged, hb - staged)

        def wait_half(h):
            @pl.when(jnp.logical_not(is_last))
            def _():
                pltpu.make_async_copy(src_hbm.at[pl.ds(b0 * EB, hb * EB)], src_all, semE).wait()
                pltpu.make_async_copy(dst_hbm.at[pl.ds(b0, hb), :], dst_all, semE).wait()
                pltpu.make_async_copy(w_hbm.at[pl.ds(b0 * EB, hb * EB)], w_all, semE).wait()

            if real_h[h] > 0:
                @pl.when(is_last)
                def _():
                    for cp in last_tail_copies(h):
                        cp.wait()

        fetch_half(0)

        zero = jnp.zeros((16,), jnp.float32)

        def zrow(r, carry):
            for j in range(LG):
                rows0[r, pl.ds(j * 16, 16)] = zero
            return carry

        lax.fori_loop(0, EB, zrow, 0)
        for k in range(nchunk):
            r0 = s * rows_per_tile + k * cchunk
            pltpu.sync_copy(rows0.at[pl.ds(0, cchunk), :],
                            accum.at[pl.ds(r0, cchunk), :])
        plsc.subcore_barrier()

        def start_gather(j, rows_r, sem):
            idx = src_all.at[pl.ds(pl.multiple_of(j * EB, EB), EB)]
            pltpu.make_async_copy(x_hbm.at[idx], rows_r, sem).start()

        def wait_gather(j, rows_r, sem):
            idx = src_all.at[pl.ds(pl.multiple_of(j * EB, EB), EB)]
            pltpu.make_async_copy(x_hbm.at[idx], rows_r, sem).wait()

        def scale(j, rows_r):
            def gbody(g, carry):
                wv = w_all[pl.ds(pl.multiple_of(j * EB + g * 16, 16), 16)]
                for el in range(16):
                    e = g * 16 + el
                    wb = lax.gather(
                        wv, jnp.full((16, 1), el, jnp.int32),
                        lax.GatherDimensionNumbers(
                            offset_dims=(), collapsed_slice_dims=(0,),
                            start_index_map=(0,)),
                        slice_sizes=(1,),
                        mode=lax.GatherScatterMode.PROMISE_IN_BOUNDS)
                    for jj in range(LG):
                        sl = pl.ds(jj * 16, 16)
                        rows_r[e, sl] = rows_r[e, sl] * wb
                return carry

            lax.fori_loop(0, EB // 16, gbody, 0)

        def scatter_add(j, rows_r):
            pltpu.sync_copy(rows_r, accum.at[dst_all.at[j]], add=True)

        pairs = hb // 2

        def loop_body(i, carry):
            j = 2 * i
            start_gather(j + 1, rows1, semG1)
            wait_gather(j, rows0, semG0)
            scale(j, rows0)
            scatter_add(j, rows0)

            @pl.when(i < pairs - 1)
            def _():
                start_gather(j + 2, rows0, semG0)

            wait_gather(j + 1, rows1, semG1)
            scale(j + 1, rows1)
            scatter_add(j + 1, rows1)
            return carry

        for h in range(2):
            if h == 1:
                fetch_half(1)
            wait_half(h)
            start_gather(0, rows0, semG0)
            lax.fori_loop(0, pairs, loop_body, 0)
        plsc.subcore_barrier()

        for k in range(nchunk):
            r0 = s * rows_per_tile + k * cchunk
            pltpu.sync_copy(accum.at[pl.ds(r0, cchunk), :],
                            out_hbm.at[c, pl.ds(r0, cchunk), :])

    mesh = plsc.VectorSubcoreMesh(core_axis_name="c", subcore_axis_name="s",
                                  num_cores=NC, num_subcores=NS)
    return pl.kernel(
        body,
        out_type=jax.ShapeDtypeStruct((NC, n_pad, d), jnp.float32),
        mesh=mesh,
        scratch_types=[
            pltpu.VMEM((bpt // 2 * EB,), jnp.int32),
            pltpu.VMEM((bpt // 2, EB), jnp.int32),
            pltpu.VMEM((bpt // 2 * EB,), jnp.float32),
            pltpu.VMEM((EB, d), jnp.float32),
            pltpu.VMEM((EB, d), jnp.float32),
            pltpu.VMEM_SHARED((n_pad, d), jnp.float32),
            pltpu.SemaphoreType.DMA,
            pltpu.SemaphoreType.DMA,
            pltpu.SemaphoreType.DMA,
        ],
    )


def _tc_body(p_ref, w_ref, b_ref, o_ref):
    acc = p_ref[0] + p_ref[1]
    o_ref[...] = (
        jnp.dot(acc, w_ref[...], preferred_element_type=jnp.float32)
        + b_ref[...]
    )


@jax.jit
def kernel(x, edge_index, edge_weight, W, b):
    n, d_in = x.shape
    d_out = W.shape[1]
    e = edge_weight.shape[0]

    src = edge_index[0].astype(jnp.int32)
    dst = edge_index[1].astype(jnp.int32)
    w = edge_weight.astype(jnp.float32)

    ept = EB * NW
    bpt = 2 * -(-e // (ept * 2))
    n_pad = -(-n // (NS * EB)) * NS * EB
    if e % EB != 0 or e // EB <= (NW - 1) * bpt:
        e_pad = bpt * ept
        npe = e_pad - e
        fill = jnp.arange(npe, dtype=jnp.int32)
        src = jnp.concatenate([src, fill % n])
        dst = jnp.concatenate([dst, n + fill % (n_pad - n)])
        w = jnp.concatenate([w, jnp.zeros((npe,), jnp.float32)])
        e = e_pad
    nb_real = e // EB

    t_blocks = nb_real - nb_real // 8 * 8
    if t_blocks > 0:
        toff = (nb_real - t_blocks) * EB
        t_pad = (8 - t_blocks) * EB
        tfill = jnp.arange(t_pad, dtype=jnp.int32)
        tsrc = jnp.concatenate([src[toff:], tfill % n])
        tdst = jnp.concatenate([dst[toff:], n + tfill % (n_pad - n)]).reshape(8, EB)
        tw = jnp.concatenate([w[toff:], jnp.zeros((t_pad,), jnp.float32)])
    else:
        tsrc = jnp.zeros((8 * EB,), jnp.int32)
        tdst = jnp.full((8, EB), n, jnp.int32)
        tw = jnp.zeros((8 * EB,), jnp.float32)

    partials = _sc_aggregate(n, n_pad, nb_real, d_in, bpt)(
        x, src, dst.reshape(nb_real, EB), w, tsrc, tdst, tw)

    rows_blk = 1000 if n % 1000 == 0 else n
    grid = n // rows_blk
    out = pl.pallas_call(
        _tc_body,
        grid=(grid,),
        in_specs=[
            pl.BlockSpec((NC, rows_blk, d_in), lambda i: (0, i, 0)),
            pl.BlockSpec((d_in, d_out), lambda i: (0, 0)),
            pl.BlockSpec((1, d_out), lambda i: (0, 0)),
        ],
        out_specs=pl.BlockSpec((rows_blk, d_out), lambda i: (i, 0)),
        out_shape=jax.ShapeDtypeStruct((n, d_out), jnp.float32),
    )(partials, W, b.reshape(1, d_out))
    return out


# speedup vs baseline: 2.0180x; 1.0063x over previous
import functools

import jax
import jax.numpy as jnp
from jax import lax
from jax.experimental import pallas as pl
from jax.experimental.pallas import tpu as pltpu
from jax.experimental.pallas import tpu_sc as plsc

NC = 2
NS = 16
NW = NC * NS
EB = 128
LG = 8


def _sc_aggregate(n, n_pad, nb_real, d, bpt):
    rows_per_tile = n_pad // NS
    cchunk = EB
    nchunk = rows_per_tile // cchunk
    hb = bpt // 2
    last_b0 = (NW - 1) * bpt
    real_h = [max(0, min(hb, nb_real - last_b0 - h * hb)) for h in range(2)]
    bulk_h = [r // 8 * 8 for r in real_h]
    tail_h = [r - b for r, b in zip(real_h, bulk_h)]
    TAIL = 8

    def body(x_hbm, src_hbm, dst_hbm, w_hbm, tsrc_hbm, tdst_hbm, tw_hbm,
             out_hbm, src_all, dst_all, w_all, rows0, rows1,
             accum, semE, semG0, semG1):
        c = lax.axis_index("c")
        s = lax.axis_index("s")
        wid = c * NS + s
        b0 = wid * bpt
        is_last = wid == NW - 1

        lanes = jax.lax.iota(jnp.int32, 16)
        wzero = jnp.zeros((16,), jnp.float32)

        def synth_fill(row_lo, nrows):
            def fr(r, carry):
                for g in range(LG):
                    sl = pl.ds(g * 16, 16)
                    fl = pl.ds(pl.multiple_of((row_lo + r) * EB + g * 16, 16), 16)
                    t = lanes * 13 + (r * 128 + g * 16)
                    src_all[fl] = t % n
                    dst_all[fl] = n + (t % (n_pad - n))
                    w_all[fl] = wzero
                return carry

            lax.fori_loop(0, nrows, fr, 0)

        def last_tail_copies(h):
            rb = bulk_h[h]
            o = b0 + h * hb
            cps = []
            if rb > 0:
                cps += [
                    pltpu.make_async_copy(
                        src_hbm.at[pl.ds(o * EB, rb * EB)],
                        src_all.at[pl.ds(0, rb * EB)], semE),
                    pltpu.make_async_copy(
                        dst_hbm.at[pl.ds(o * EB, rb * EB)],
                        dst_all.at[pl.ds(0, rb * EB)], semE),
                    pltpu.make_async_copy(
                        w_hbm.at[pl.ds(o * EB, rb * EB)],
                        w_all.at[pl.ds(0, rb * EB)], semE),
                ]
            if tail_h[h] > 0:
                cps += [
                    pltpu.make_async_copy(
                        tsrc_hbm, src_all.at[pl.ds(rb * EB, TAIL * EB)], semE),
                    pltpu.make_async_copy(
                        tdst_hbm, dst_all.at[pl.ds(rb * EB, TAIL * EB)], semE),
                    pltpu.make_async_copy(
                        tw_hbm, w_all.at[pl.ds(rb * EB, TAIL * EB)], semE),
                ]
            return cps

        def fetch_half(h):
            o = b0 + h * hb

            @pl.when(jnp.logical_not(is_last))
            def _():
                pltpu.make_async_copy(src_hbm.at[pl.ds(o * EB, hb * EB)], src_all, semE).start()
                pltpu.make_async_copy(dst_hbm.at[pl.ds(o * EB, hb * EB)], dst_all, semE).start()
                pltpu.make_async_copy(w_hbm.at[pl.ds(o * EB, hb * EB)], w_all, semE).start()

            @pl.when(is_last)
            def _():
                for cp in last_tail_copies(h):
                    cp.start()
                staged = bulk_h[h] + (TAIL if tail_h[h] > 0 else 0)
                if staged < hb:
                    synth_fill(staged, hb - staged)

        def wait_half(h):
            @pl.when(jnp.logical_not(is_last))
            def _():
                pltpu.make_async_copy(src_hbm.at[pl.ds(b0 * EB, hb * EB)], src_all, semE).wait()
                pltpu.make_async_copy(dst_hbm.at[pl.ds(b0 * EB, hb * EB)], dst_all, semE).wait()
                pltpu.make_async_copy(w_hbm.at[pl.ds(b0 * EB, hb * EB)], w_all, semE).wait()

            if real_h[h] > 0:
                @pl.when(is_last)
                def _():
                    for cp in last_tail_copies(h):
                        cp.wait()

        fetch_half(0)

        zero = jnp.zeros((16,), jnp.float32)

        def zrow(r, carry):
            for j in range(LG):
                rows0[r, pl.ds(j * 16, 16)] = zero
            return carry

        lax.fori_loop(0, EB, zrow, 0)
        for k in range(nchunk):
            r0 = s * rows_per_tile + k * cchunk
            pltpu.sync_copy(rows0.at[pl.ds(0, cchunk), :],
                            accum.at[pl.ds(r0, cchunk), :])
        plsc.subcore_barrier()

        def start_gather(j, rows_r, sem):
            idx = src_all.at[pl.ds(pl.multiple_of(j * EB, EB), EB)]
            pltpu.make_async_copy(x_hbm.at[idx], rows_r, sem).start()

        def wait_gather(j, rows_r, sem):
            idx = src_all.at[pl.ds(pl.multiple_of(j * EB, EB), EB)]
            pltpu.make_async_copy(x_hbm.at[idx], rows_r, sem).wait()

        def scale(j, rows_r):
            def gbody(g, carry):
                wv = w_all[pl.ds(pl.multiple_of(j * EB + g * 16, 16), 16)]
                for el in range(16):
                    e = g * 16 + el
                    wb = lax.gather(
                        wv, jnp.full((16, 1), el, jnp.int32),
                        lax.GatherDimensionNumbers(
                            offset_dims=(), collapsed_slice_dims=(0,),
                            start_index_map=(0,)),
                        slice_sizes=(1,),
                        mode=lax.GatherScatterMode.PROMISE_IN_BOUNDS)
                    for jj in range(LG):
                        sl = pl.ds(jj * 16, 16)
                        rows_r[e, sl] = rows_r[e, sl] * wb
                return carry

            lax.fori_loop(0, EB // 16, gbody, 0)

        def scatter_add(j, rows_r):
            idx = dst_all.at[pl.ds(pl.multiple_of(j * EB, EB), EB)]
            pltpu.sync_copy(rows_r, accum.at[idx], add=True)

        pairs = hb // 2

        def loop_body(i, carry):
            j = 2 * i
            start_gather(j + 1, rows1, semG1)
            wait_gather(j, rows0, semG0)
            scale(j, rows0)
            scatter_add(j, rows0)

            @pl.when(i < pairs - 1)
            def _():
                start_gather(j + 2, rows0, semG0)

            wait_gather(j + 1, rows1, semG1)
            scale(j + 1, rows1)
            scatter_add(j + 1, rows1)
            return carry

        for h in range(2):
            if h == 1:
                fetch_half(1)
            wait_half(h)
            start_gather(0, rows0, semG0)
            lax.fori_loop(0, pairs, loop_body, 0)
        plsc.subcore_barrier()

        for k in range(nchunk):
            r0 = s * rows_per_tile + k * cchunk
            pltpu.sync_copy(accum.at[pl.ds(r0, cchunk), :],
                            out_hbm.at[c, pl.ds(r0, cchunk), :])

    mesh = plsc.VectorSubcoreMesh(core_axis_name="c", subcore_axis_name="s",
                                  num_cores=NC, num_subcores=NS)
    return pl.kernel(
        body,
        out_type=jax.ShapeDtypeStruct((NC, n_pad, d), jnp.float32),
        mesh=mesh,
        scratch_types=[
            pltpu.VMEM((bpt // 2 * EB,), jnp.int32),
            pltpu.VMEM((bpt // 2 * EB,), jnp.int32),
            pltpu.VMEM((bpt // 2 * EB,), jnp.float32),
            pltpu.VMEM((EB, d), jnp.float32),
            pltpu.VMEM((EB, d), jnp.float32),
            pltpu.VMEM_SHARED((n_pad, d), jnp.float32),
            pltpu.SemaphoreType.DMA,
            pltpu.SemaphoreType.DMA,
            pltpu.SemaphoreType.DMA,
        ],
    )


def _tc_body(p_ref, w_ref, b_ref, o_ref):
    acc = p_ref[0] + p_ref[1]
    o_ref[...] = (
        jnp.dot(acc, w_ref[...], preferred_element_type=jnp.float32)
        + b_ref[...]
    )


@jax.jit
def kernel(x, edge_index, edge_weight, W, b):
    n, d_in = x.shape
    d_out = W.shape[1]
    e = edge_weight.shape[0]

    src = edge_index[0].astype(jnp.int32)
    dst = edge_index[1].astype(jnp.int32)
    w = edge_weight.astype(jnp.float32)

    ept = EB * NW
    bpt = 2 * -(-e // (ept * 2))
    n_pad = -(-n // (NS * EB)) * NS * EB
    if e % EB != 0 or e // EB <= (NW - 1) * bpt:
        e_pad = bpt * ept
        npe = e_pad - e
        fill = jnp.arange(npe, dtype=jnp.int32)
        src = jnp.concatenate([src, fill % n])
        dst = jnp.concatenate([dst, n + fill % (n_pad - n)])
        w = jnp.concatenate([w, jnp.zeros((npe,), jnp.float32)])
        e = e_pad
    nb_real = e // EB

    t_blocks = nb_real - nb_real // 8 * 8
    if t_blocks > 0:
        toff = (nb_real - t_blocks) * EB
        t_pad = (8 - t_blocks) * EB
        tfill = jnp.arange(t_pad, dtype=jnp.int32)
        tsrc = jnp.concatenate([src[toff:], tfill % n])
        tdst = jnp.concatenate([dst[toff:], n + tfill % (n_pad - n)])
        tw = jnp.concatenate([w[toff:], jnp.zeros((t_pad,), jnp.float32)])
    else:
        tsrc = jnp.zeros((8 * EB,), jnp.int32)
        tdst = jnp.full((8 * EB,), n, jnp.int32)
        tw = jnp.zeros((8 * EB,), jnp.float32)

    partials = _sc_aggregate(n, n_pad, nb_real, d_in, bpt)(
        x, src, dst, w, tsrc, tdst, tw)

    rows_blk = 1000 if n % 1000 == 0 else n
    grid = n // rows_blk
    out = pl.pallas_call(
        _tc_body,
        grid=(grid,),
        in_specs=[
            pl.BlockSpec((NC, rows_blk, d_in), lambda i: (0, i, 0)),
            pl.BlockSpec((d_in, d_out), lambda i: (0, 0)),
            pl.BlockSpec((1, d_out), lambda i: (0, 0)),
        ],
        out_specs=pl.BlockSpec((rows_blk, d_out), lambda i: (i, 0)),
        out_shape=jax.ShapeDtypeStruct((n, d_out), jnp.float32),
    )(partials, W, b.reshape(1, d_out))
    return out


# speedup vs baseline: 2.0222x; 1.0021x over previous
import jax
import jax.numpy as jnp
from jax import lax
from jax.experimental import pallas as pl
from jax.experimental.pallas import tpu as pltpu
from jax.experimental.pallas import tpu_sc as plsc

NC = 2
NS = 16
NW = NC * NS
EB = 128
LG = 8


def _sc_aggregate(n, n_pad, nb_real, d, bpt):
    rows_per_tile = n_pad // NS
    cchunk = EB
    nchunk = rows_per_tile // cchunk
    hb = bpt // 2
    last_b0 = (NW - 1) * bpt
    real_h = [max(0, min(hb, nb_real - last_b0 - h * hb)) for h in range(2)]
    bulk_h = [r // 8 * 8 for r in real_h]
    tail_h = [r - b for r, b in zip(real_h, bulk_h)]
    TAIL = 8

    def body(x_hbm, src_hbm, dst_hbm, w_hbm, tsrc_hbm, tdst_hbm, tw_hbm,
             out_hbm, src_all, dst_all, w_all, rows0, rows1,
             accum, semE, semG0, semG1):
        c = lax.axis_index("c")
        s = lax.axis_index("s")
        wid = c * NS + s
        b0 = wid * bpt
        is_last = wid == NW - 1

        lanes = jax.lax.iota(jnp.int32, 16)
        wzero = jnp.zeros((16,), jnp.float32)

        def synth_fill(row_lo, nrows):
            def fr(r, carry):
                for g in range(LG):
                    sl = pl.ds(g * 16, 16)
                    fl = pl.ds(pl.multiple_of((row_lo + r) * EB + g * 16, 16), 16)
                    t = lanes * 13 + (r * 128 + g * 16)
                    src_all[fl] = t % n
                    dst_all[fl] = n + (t % (n_pad - n))
                    w_all[fl] = wzero
                return carry

            lax.fori_loop(0, nrows, fr, 0)

        def last_tail_copies(h):
            rb = bulk_h[h]
            o = b0 + h * hb
            cps = []
            if rb > 0:
                cps += [
                    pltpu.make_async_copy(
                        src_hbm.at[pl.ds(o * EB, rb * EB)],
                        src_all.at[pl.ds(0, rb * EB)], semE),
                    pltpu.make_async_copy(
                        dst_hbm.at[pl.ds(o * EB, rb * EB)],
                        dst_all.at[pl.ds(0, rb * EB)], semE),
                    pltpu.make_async_copy(
                        w_hbm.at[pl.ds(o * EB, rb * EB)],
                        w_all.at[pl.ds(0, rb * EB)], semE),
                ]
            if tail_h[h] > 0:
                cps += [
                    pltpu.make_async_copy(
                        tsrc_hbm, src_all.at[pl.ds(rb * EB, TAIL * EB)], semE),
                    pltpu.make_async_copy(
                        tdst_hbm, dst_all.at[pl.ds(rb * EB, TAIL * EB)], semE),
                    pltpu.make_async_copy(
                        tw_hbm, w_all.at[pl.ds(rb * EB, TAIL * EB)], semE),
                ]
            return cps

        def fetch_half(h):
            o = b0 + h * hb

            @pl.when(jnp.logical_not(is_last))
            def _():
                pltpu.make_async_copy(src_hbm.at[pl.ds(o * EB, hb * EB)], src_all, semE).start()
                pltpu.make_async_copy(dst_hbm.at[pl.ds(o * EB, hb * EB)], dst_all, semE).start()
                pltpu.make_async_copy(w_hbm.at[pl.ds(o * EB, hb * EB)], w_all, semE).start()

            @pl.when(is_last)
            def _():
                for cp in last_tail_copies(h):
                    cp.start()
                staged = bulk_h[h] + (TAIL if tail_h[h] > 0 else 0)
                if staged < hb:
                    synth_fill(staged, hb - staged)

        def wait_half(h):
            @pl.when(jnp.logical_not(is_last))
            def _():
                pltpu.make_async_copy(src_hbm.at[pl.ds(b0 * EB, hb * EB)], src_all, semE).wait()
                pltpu.make_async_copy(dst_hbm.at[pl.ds(b0 * EB, hb * EB)], dst_all, semE).wait()
                pltpu.make_async_copy(w_hbm.at[pl.ds(b0 * EB, hb * EB)], w_all, semE).wait()

            if real_h[h] > 0:
                @pl.when(is_last)
                def _():
                    for cp in last_tail_copies(h):
                        cp.wait()

        fetch_half(0)

        zero = jnp.zeros((16,), jnp.float32)

        def zrow(r, carry):
            for j in range(LG):
                rows0[r, pl.ds(j * 16, 16)] = zero
            return carry

        lax.fori_loop(0, EB, zrow, 0)
        for k in range(nchunk):
            r0 = s * rows_per_tile + k * cchunk
            pltpu.sync_copy(rows0.at[pl.ds(0, cchunk), :],
                            accum.at[pl.ds(r0, cchunk), :])
        plsc.subcore_barrier()

        def start_gather(j, rows_r, sem):
            idx = src_all.at[pl.ds(pl.multiple_of(j * EB, EB), EB)]
            pltpu.make_async_copy(x_hbm.at[idx], rows_r, sem).start()

        def wait_gather(j, rows_r, sem):
            idx = src_all.at[pl.ds(pl.multiple_of(j * EB, EB), EB)]
            pltpu.make_async_copy(x_hbm.at[idx], rows_r, sem).wait()

        def scale(j, rows_r):
            def gbody(g, carry):
                wv = w_all[pl.ds(pl.multiple_of(j * EB + g * 16, 16), 16)]
                for el in range(16):
                    e = g * 16 + el
                    wb = lax.gather(
                        wv, jnp.full((16, 1), el, jnp.int32),
                        lax.GatherDimensionNumbers(
                            offset_dims=(), collapsed_slice_dims=(0,),
                            start_index_map=(0,)),
                        slice_sizes=(1,),
                        mode=lax.GatherScatterMode.PROMISE_IN_BOUNDS)
                    for jj in range(LG):
                        sl = pl.ds(jj * 16, 16)
                        rows_r[e, sl] = rows_r[e, sl] * wb
                return carry

            lax.fori_loop(0, EB // 16, gbody, 0)

        def scatter_add(j, rows_r):
            idx = dst_all.at[pl.ds(pl.multiple_of(j * EB, EB), EB)]
            pltpu.sync_copy(rows_r, accum.at[idx], add=True)

        pairs = hb // 2

        def loop_body(i, carry):
            j = 2 * i
            start_gather(j + 1, rows1, semG1)
            wait_gather(j, rows0, semG0)
            scale(j, rows0)
            scatter_add(j, rows0)

            @pl.when(i < pairs - 1)
            def _():
                start_gather(j + 2, rows0, semG0)

            wait_gather(j + 1, rows1, semG1)
            scale(j + 1, rows1)
            scatter_add(j + 1, rows1)
            return carry

        for h in range(2):
            if h == 1:
                fetch_half(1)
            wait_half(h)
            start_gather(0, rows0, semG0)
            lax.fori_loop(0, pairs, loop_body, 0)
        plsc.subcore_barrier()

        for k in range(nchunk):
            r0 = s * rows_per_tile + k * cchunk
            pltpu.sync_copy(accum.at[pl.ds(r0, cchunk), :],
                            out_hbm.at[c, pl.ds(r0, cchunk), :])

    mesh = plsc.VectorSubcoreMesh(core_axis_name="c", subcore_axis_name="s",
                                  num_cores=NC, num_subcores=NS)
    return pl.kernel(
        body,
        out_type=jax.ShapeDtypeStruct((NC, n_pad, d), jnp.float32),
        mesh=mesh,
        scratch_types=[
            pltpu.VMEM((bpt // 2 * EB,), jnp.int32),
            pltpu.VMEM((bpt // 2 * EB,), jnp.int32),
            pltpu.VMEM((bpt // 2 * EB,), jnp.float32),
            pltpu.VMEM((EB, d), jnp.float32),
            pltpu.VMEM((EB, d), jnp.float32),
            pltpu.VMEM_SHARED((n_pad, d), jnp.float32),
            pltpu.SemaphoreType.DMA,
            pltpu.SemaphoreType.DMA,
            pltpu.SemaphoreType.DMA,
        ],
    )


def _tc_body(p_ref, w_ref, b_ref, o_ref):
    acc = p_ref[0] + p_ref[1]
    o_ref[...] = (
        jnp.dot(acc, w_ref[...], preferred_element_type=jnp.float32)
        + b_ref[...]
    )


@jax.jit
def kernel(x, edge_index, edge_weight, W, b):
    n, d_in = x.shape
    d_out = W.shape[1]
    e = edge_weight.shape[0]

    src = edge_index[0].astype(jnp.int32)
    dst = edge_index[1].astype(jnp.int32)
    w = edge_weight.astype(jnp.float32)

    ept = EB * NW
    bpt = 2 * -(-e // (ept * 2))
    n_pad = -(-n // (NS * EB)) * NS * EB
    if e % EB != 0 or e // EB <= (NW - 1) * bpt:
        e_pad = bpt * ept
        npe = e_pad - e
        fill = jnp.arange(npe, dtype=jnp.int32)
        src = jnp.concatenate([src, fill % n])
        dst = jnp.concatenate([dst, n + fill % (n_pad - n)])
        w = jnp.concatenate([w, jnp.zeros((npe,), jnp.float32)])
        e = e_pad
    nb_real = e // EB

    t_blocks = nb_real - nb_real // 8 * 8
    if t_blocks > 0:
        toff = (nb_real - t_blocks) * EB
        t_pad = (8 - t_blocks) * EB
        tfill = jnp.arange(t_pad, dtype=jnp.int32)
        tsrc = jnp.concatenate([src[toff:], tfill % n])
        tdst = jnp.concatenate([dst[toff:], n + tfill % (n_pad - n)])
        tw = jnp.concatenate([w[toff:], jnp.zeros((t_pad,), jnp.float32)])
    else:
        tsrc = jnp.zeros((8 * EB,), jnp.int32)
        tdst = jnp.full((8 * EB,), n, jnp.int32)
        tw = jnp.zeros((8 * EB,), jnp.float32)

    partials = _sc_aggregate(n, n_pad, nb_real, d_in, bpt)(
        x, src, dst, w, tsrc, tdst, tw)

    rows_blk = 1000 if n % 1000 == 0 else n
    grid = n // rows_blk
    out = pl.pallas_call(
        _tc_body,
        grid=(grid,),
        in_specs=[
            pl.BlockSpec((NC, rows_blk, d_in), lambda i: (0, i, 0)),
            pl.BlockSpec((d_in, d_out), lambda i: (0, 0)),
            pl.BlockSpec((1, d_out), lambda i: (0, 0)),
        ],
        out_specs=pl.BlockSpec((rows_blk, d_out), lambda i: (i, 0)),
        out_shape=jax.ShapeDtypeStruct((n, d_out), jnp.float32),
    )(partials, W, b.reshape(1, d_out))
    return out
